# baseline (speedup 1.0000x reference)
import functools

import jax
import jax.numpy as jnp
from jax import lax
from jax.experimental import pallas as pl
from jax.experimental.pallas import tpu as pltpu
from jax.experimental.pallas import tpu_sc as plsc

N_TYPES = 100000
SEQ_LEN = 200
TOTAL = SEQ_LEN * N_TYPES
NUM_CORES = 2
NUM_SUBCORES = 16
NW = NUM_CORES * NUM_SUBCORES
REGION = TOTAL // NW
CHUNK = 25_000
N_CHUNKS = REGION // CHUNK
ZBUF = 25_088
LANES = 16
SEQ_PAD = 224

_mesh = plsc.VectorSubcoreMesh(core_axis_name="c", subcore_axis_name="s")


@functools.partial(
    pl.kernel,
    out_type=jax.ShapeDtypeStruct((TOTAL,), jnp.float32),
    mesh=_mesh,
    scratch_types=[
        pltpu.VMEM((ZBUF,), jnp.float32),
        pltpu.VMEM((SEQ_PAD,), jnp.int32),
        pltpu.VMEM((LANES,), jnp.int32),
        pltpu.VMEM((LANES,), jnp.float32),
        pltpu.SemaphoreType.DMA,
        pltpu.SemaphoreType.DMA,
    ],
    compiler_params=pltpu.CompilerParams(needs_layout_passes=False),
)
def _bow_sc(tokens_hbm, scale_hbm, out_hbm, zbuf, tbuf, ibuf, vbuf, zsem, ssem):
    wid = lax.axis_index("c") * NUM_SUBCORES + lax.axis_index("s")
    lo = pl.multiple_of(wid * REGION, 8)

    zv = jnp.zeros((LANES,), jnp.float32)

    def zbody(i, carry):
        b = i * (8 * LANES)
        for k in range(8):
            zbuf[pl.ds(b + k * LANES, LANES)] = zv
        return carry

    lax.fori_loop(0, ZBUF // (8 * LANES), zbody, 0)

    copies = []
    for j in range(N_CHUNKS):
        cp = pltpu.make_async_copy(
            zbuf.at[pl.ds(0, CHUNK)],
            out_hbm.at[pl.ds(lo + j * CHUNK, CHUNK)],
            zsem,
        )
        cp.start()
        copies.append(cp)

    pltpu.sync_copy(tokens_hbm, tbuf)
    pltpu.sync_copy(scale_hbm, vbuf)

    base = (wid * SEQ_LEN) // NW
    r = base + lax.iota(jnp.int32, LANES)
    tok = tbuf[pl.ds(base, LANES)]
    flat = r * N_TYPES + tok
    mask = (flat >= lo) & (flat < lo + REGION)
    mx = jnp.max(jnp.where(mask, flat, -1))
    ibuf[...] = jnp.where(mask, flat, mx)

    for cp in copies:
        cp.wait()
    pltpu.async_copy(vbuf, out_hbm.at[ibuf], ssem).wait()


def kernel(tokens, scale):
    tokens32 = jnp.pad(tokens.astype(jnp.int32), (0, SEQ_PAD - SEQ_LEN))
    scale16 = jnp.broadcast_to(scale.astype(jnp.float32), (LANES,))
    flat = _bow_sc(tokens32, scale16)
    return flat.reshape(SEQ_LEN, 1, N_TYPES)
